# Initial kernel scaffold; baseline (speedup 1.0000x reference)
import jax
import jax.numpy as jnp
from jax.experimental import pallas as pl


def kernel(x, table):
    raise NotImplementedError("write your pallas kernel here")


# speedup vs baseline: 3.6647x; 3.6647x over previous
import functools

import jax
import jax.numpy as jnp
from jax import lax
from jax.experimental import pallas as pl
from jax.experimental.pallas import tpu as pltpu
from jax.experimental.pallas import tpu_sc as plsc

_CHUNK = 128


@functools.lru_cache(maxsize=None)
def _build(total, dim):
    info = plsc.get_sparse_core_info()
    num_workers = info.num_cores * info.num_subcores
    per_worker = total // num_workers
    assert per_worker * num_workers == total
    n_chunks = per_worker // _CHUNK
    assert n_chunks * _CHUNK == per_worker
    mesh = plsc.VectorSubcoreMesh(core_axis_name="c", subcore_axis_name="s")

    @functools.partial(
        pl.kernel,
        mesh=mesh,
        out_type=jax.ShapeDtypeStruct((total, dim), jnp.float32),
        scratch_types=[
            pltpu.VMEM((_CHUNK,), jnp.int32),
            pltpu.VMEM((_CHUNK, dim), jnp.float32),
            pltpu.SemaphoreType.DMA,
        ],
        compiler_params=pltpu.CompilerParams(use_tc_tiling_on_sc=False),
    )
    def lookup(x_hbm, table_hbm, out_hbm, idx_v, rows_v, sem):
        wid = lax.axis_index("s") * info.num_cores + lax.axis_index("c")
        base = wid * per_worker

        def body(j, carry):
            off = base + j * _CHUNK
            pltpu.sync_copy(x_hbm.at[pl.ds(off, _CHUNK)], idx_v)
            pltpu.async_copy(table_hbm.at[idx_v], rows_v, sem).wait()
            pltpu.sync_copy(rows_v, out_hbm.at[pl.ds(off, _CHUNK)])
            return carry

        lax.fori_loop(0, n_chunks, body, 0)

    return lookup


def kernel(x, table):
    batch, hist = x.shape
    dim = table.shape[1]
    flat = x.reshape(-1).astype(jnp.int32)
    out = _build(flat.shape[0], dim)(flat, table.astype(jnp.float32))
    return out.reshape(batch, hist, dim)


# speedup vs baseline: 4.1640x; 1.1363x over previous
import functools

import jax
import jax.numpy as jnp
from jax import lax
from jax.experimental import pallas as pl
from jax.experimental.pallas import tpu as pltpu
from jax.experimental.pallas import tpu_sc as plsc

_G = 128
_NG = 5
_SUP = _G * _NG


@functools.lru_cache(maxsize=None)
def _build(total, dim):
    info = plsc.get_sparse_core_info()
    num_workers = info.num_cores * info.num_subcores
    per_worker = total // num_workers
    assert per_worker * num_workers == total
    n_sup = per_worker // _SUP
    assert n_sup * _SUP == per_worker and n_sup >= 4 and n_sup % 2 == 0
    mesh = plsc.VectorSubcoreMesh(core_axis_name="c", subcore_axis_name="s")

    @functools.partial(
        pl.kernel,
        mesh=mesh,
        out_type=jax.ShapeDtypeStruct((total, dim), jnp.float32),
        scratch_types=[
            pltpu.VMEM((2, _SUP), jnp.int32),
            pltpu.VMEM((2, _SUP, dim), jnp.float32),
            pltpu.SemaphoreType.DMA,
            pltpu.SemaphoreType.DMA,
            pltpu.SemaphoreType.DMA,
            pltpu.SemaphoreType.DMA,
            pltpu.SemaphoreType.DMA,
            pltpu.SemaphoreType.DMA,
        ],
        compiler_params=pltpu.CompilerParams(use_tc_tiling_on_sc=False),
    )
    def lookup(x_hbm, table_hbm, out_hbm, idx_v, rows_v, si0, si1, sg0, sg1,
               so0, so1):
        sems_i = (si0, si1)
        sems_g = (sg0, sg1)
        sems_o = (so0, so1)
        wid = lax.axis_index("s") * info.num_cores + lax.axis_index("c")
        base = wid * per_worker

        def idx_copy(s, b):
            return pltpu.make_async_copy(
                x_hbm.at[pl.ds(base + s * _SUP, _SUP)], idx_v.at[b], sems_i[b])

        def out_copy(s, b):
            return pltpu.make_async_copy(
                rows_v.at[b], out_hbm.at[pl.ds(base + s * _SUP, _SUP)],
                sems_o[b])

        def gather(b):
            for k in range(_NG):
                pltpu.make_async_copy(
                    table_hbm.at[idx_v.at[b, pl.ds(k * _G, _G)]],
                    rows_v.at[b, pl.ds(k * _G, _G)], sems_g[b]).start()
            for k in range(_NG):
                pltpu.make_async_copy(
                    table_hbm.at[idx_v.at[b, pl.ds(k * _G, _G)]],
                    rows_v.at[b, pl.ds(k * _G, _G)], sems_g[b]).wait()

        idx_copy(0, 0).start()
        idx_copy(1, 1).start()
        for b in range(2):
            idx_copy(b, b).wait()
            gather(b)
            idx_copy(b + 2, b).start()
            out_copy(b, b).start()

        def body(t, carry):
            s0 = 2 * t
            for b in range(2):
                s = s0 + b
                idx_copy(s, b).wait()
                out_copy(s - 2, b).wait()
                gather(b)
                idx_copy(s + 2, b).start()
                out_copy(s, b).start()
            return carry

        lax.fori_loop(1, n_sup // 2 - 1, body, 0)

        for b in range(2):
            s = n_sup - 2 + b
            idx_copy(s, b).wait()
            out_copy(s - 2, b).wait()
            gather(b)
            out_copy(s, b).start()
        for b in range(2):
            out_copy(n_sup - 2 + b, b).wait()

    return lookup


def kernel(x, table):
    batch, hist = x.shape
    dim = table.shape[1]
    flat = x.reshape(-1).astype(jnp.int32)
    out = _build(flat.shape[0], dim)(flat, table.astype(jnp.float32))
    return out.reshape(batch, hist, dim)


# speedup vs baseline: 5.8048x; 1.3941x over previous
import functools

import jax
import jax.numpy as jnp
from jax import lax
from jax.experimental import pallas as pl
from jax.experimental.pallas import tpu as pltpu
from jax.experimental.pallas import tpu_sc as plsc

_G = 128
_NG = 5
_SUP = _G * _NG


@functools.lru_cache(maxsize=None)
def _build(total, rows, dim):
    info = plsc.get_sparse_core_info()
    num_workers = info.num_cores * info.num_subcores
    per_worker = total // num_workers
    assert per_worker * num_workers == total
    n_sup = per_worker // _SUP
    assert n_sup * _SUP == per_worker and n_sup >= 4 and n_sup % 2 == 0
    mesh = plsc.VectorSubcoreMesh(core_axis_name="c", subcore_axis_name="s")

    @functools.partial(
        pl.kernel,
        mesh=mesh,
        out_type=jax.ShapeDtypeStruct((total, dim), jnp.float32),
        scratch_types=[
            pltpu.VMEM_SHARED((rows, dim), jnp.float32),
            pltpu.VMEM((2, _SUP), jnp.int32),
            pltpu.VMEM((2, _SUP, dim), jnp.float32),
            pltpu.SemaphoreType.DMA,
            pltpu.SemaphoreType.DMA,
            pltpu.SemaphoreType.DMA,
            pltpu.SemaphoreType.DMA,
            pltpu.SemaphoreType.DMA,
            pltpu.SemaphoreType.DMA,
        ],
        compiler_params=pltpu.CompilerParams(use_tc_tiling_on_sc=False),
    )
    def lookup(x_hbm, table_hbm, out_hbm, table_sh, idx_v, rows_v, si0, si1,
               sg0, sg1, so0, so1):
        sems_i = (si0, si1)
        sems_g = (sg0, sg1)
        sems_o = (so0, so1)
        sid = lax.axis_index("s")
        wid = sid * info.num_cores + lax.axis_index("c")
        base = wid * per_worker

        @pl.when(sid == 0)
        def _():
            pltpu.sync_copy(table_hbm, table_sh)

        plsc.subcore_barrier()

        def idx_copy(s, b):
            return pltpu.make_async_copy(
                x_hbm.at[pl.ds(base + s * _SUP, _SUP)], idx_v.at[b], sems_i[b])

        def out_copy(s, b):
            return pltpu.make_async_copy(
                rows_v.at[b], out_hbm.at[pl.ds(base + s * _SUP, _SUP)],
                sems_o[b])

        def gather(b):
            for k in range(_NG):
                pltpu.make_async_copy(
                    table_sh.at[idx_v.at[b, pl.ds(k * _G, _G)]],
                    rows_v.at[b, pl.ds(k * _G, _G)], sems_g[b]).start()
            for k in range(_NG):
                pltpu.make_async_copy(
                    table_sh.at[idx_v.at[b, pl.ds(k * _G, _G)]],
                    rows_v.at[b, pl.ds(k * _G, _G)], sems_g[b]).wait()

        idx_copy(0, 0).start()
        idx_copy(1, 1).start()
        for b in range(2):
            idx_copy(b, b).wait()
            gather(b)
            idx_copy(b + 2, b).start()
            out_copy(b, b).start()

        def body(t, carry):
            s0 = 2 * t
            for b in range(2):
                s = s0 + b
                idx_copy(s, b).wait()
                out_copy(s - 2, b).wait()
                gather(b)
                idx_copy(s + 2, b).start()
                out_copy(s, b).start()
            return carry

        lax.fori_loop(1, n_sup // 2 - 1, body, 0)

        for b in range(2):
            s = n_sup - 2 + b
            idx_copy(s, b).wait()
            out_copy(s - 2, b).wait()
            gather(b)
            out_copy(s, b).start()
        for b in range(2):
            out_copy(n_sup - 2 + b, b).wait()

    return lookup


def kernel(x, table):
    batch, hist = x.shape
    dim = table.shape[1]
    flat = x.reshape(-1).astype(jnp.int32)
    out = _build(flat.shape[0], table.shape[0], dim)(
        flat, table.astype(jnp.float32))
    return out.reshape(batch, hist, dim)
